# Initial kernel scaffold; baseline (speedup 1.0000x reference)
import jax
import jax.numpy as jnp
from jax.experimental import pallas as pl


def kernel(x, table):
    raise NotImplementedError("write your pallas kernel here")


# speedup vs baseline: 3.7806x; 3.7806x over previous
import functools

import jax
import jax.numpy as jnp
import numpy as np
from jax import lax
from jax.experimental import pallas as pl
from jax.experimental.pallas import tpu as pltpu
from jax.experimental.pallas import tpu_sc as plsc

NUM_FIELDS = 26
FIELD_SIZE = 1000
EMBED_DIM = 64
BATCH = 4096

NC, NS, LANES = 2, 16, 16
NW = NC * NS
TOTAL = BATCH * NUM_FIELDS
B_PER_W = TOTAL // NW
N_CHUNKS = 4
CHUNK = B_PER_W // N_CHUNKS

_FIELD_OFFSETS = np.arange(NUM_FIELDS, dtype=np.int32) * FIELD_SIZE
_TILED_OFFSETS = np.tile(_FIELD_OFFSETS, B_PER_W // NUM_FIELDS)


def _sc_body(x_hbm, off_hbm, table_hbm, out_hbm, idx_v, off_v, rows_v,
             sem0, sem1):
    wid = lax.axis_index("s") * NC + lax.axis_index("c")
    base = wid * B_PER_W

    pltpu.sync_copy(x_hbm.at[pl.ds(base, B_PER_W)], idx_v)
    pltpu.sync_copy(off_hbm, off_v)

    def add_body(i, carry):
        s = pl.ds(pl.multiple_of(i * LANES, LANES), LANES)
        idx_v[s] = idx_v[s] + off_v[s]
        return carry

    lax.fori_loop(0, B_PER_W // LANES, add_body, 0, unroll=8)

    sems = (sem0, sem1)
    descs = [None, None]
    descs[0] = pltpu.async_copy(
        table_hbm.at[idx_v.at[pl.ds(0, CHUNK)]], rows_v.at[0], sems[0])
    for c in range(N_CHUNKS):
        b = c % 2
        if c + 1 < N_CHUNKS:
            nb = (c + 1) % 2
            descs[nb] = pltpu.async_copy(
                table_hbm.at[idx_v.at[pl.ds((c + 1) * CHUNK, CHUNK)]],
                rows_v.at[nb], sems[nb])
        descs[b].wait()
        pltpu.sync_copy(rows_v.at[b],
                        out_hbm.at[pl.ds(base + c * CHUNK, CHUNK)])


@jax.jit
def _embed(x_flat, offsets, table):
    mesh = plsc.VectorSubcoreMesh(
        core_axis_name="c", subcore_axis_name="s",
        num_cores=NC, num_subcores=NS)
    run = pl.kernel(
        _sc_body,
        out_type=jax.ShapeDtypeStruct((TOTAL, EMBED_DIM), jnp.float32),
        mesh=mesh,
        scratch_types=[
            pltpu.VMEM((B_PER_W,), jnp.int32),
            pltpu.VMEM((B_PER_W,), jnp.int32),
            pltpu.VMEM((2, CHUNK, EMBED_DIM), jnp.float32),
            pltpu.SemaphoreType.DMA,
            pltpu.SemaphoreType.DMA,
        ],
        compiler_params=pltpu.CompilerParams(use_tc_tiling_on_sc=False),
    )
    return run(x_flat, offsets, table)


def kernel(x, table):
    x_flat = x.reshape(TOTAL)
    out = _embed(x_flat, jnp.asarray(_TILED_OFFSETS), table)
    return out.reshape(BATCH, NUM_FIELDS, EMBED_DIM)
